# Initial kernel scaffold; baseline (speedup 1.0000x reference)
import jax
import jax.numpy as jnp
from jax import lax
from jax.experimental import pallas as pl
from jax.experimental.pallas import tpu as pltpu
from jax.experimental.pallas import tpu_sc as plsc

ENT_MAX_LEN = 20
REL_MAX_LEN = 20
DIM = 128
BATCH = 4096

NUM_CORES = 2
NUM_SUBCORES = 16
NW = NUM_CORES * NUM_SUBCORES
BW = BATCH // NW
NBUF = 4
L = 16


def _accum_rows(buf, n_rows, acc, i, col0):
  for c in range(DIM // L):
    s = buf[0, pl.ds(c * L, L)]
    for t in range(1, n_rows):
      s = s + buf[t, pl.ds(c * L, L)]
    acc[i, pl.ds(col0 + c * L, L)] = s


def _pooled_side(tok_v, emb_h, n_tok, acc_v, col0, bufs, sems):
  n_grp = BW // NBUF

  def grp(g, _):
    i0 = g * NBUF
    copies = []
    for b in range(NBUF):
      cp = pltpu.async_copy(emb_h.at[tok_v.at[i0 + b]], bufs[b], sems[b])
      copies.append(cp)
    for b in range(NBUF):
      copies[b].wait()
      _accum_rows(bufs[b], n_tok, acc_v, i0 + b, col0)
    return 0

  lax.fori_loop(0, n_grp, grp, 0)


def _body(subj_h, rel_h, etok_h, rtok_h, eemb_h, remb_h, out_h,
          sidx_v, ridx_v, etok_v, rtok_v, acc_v, bufs, sems, idx_sem):
  c = lax.axis_index("c")
  s = lax.axis_index("s")
  wid = s * NUM_CORES + c
  base = wid * BW

  pltpu.sync_copy(subj_h.at[pl.ds(base, BW)], sidx_v)
  pltpu.sync_copy(rel_h.at[pl.ds(base, BW)], ridx_v)
  pltpu.async_copy(etok_h.at[sidx_v], etok_v, idx_sem).wait()
  pltpu.async_copy(rtok_h.at[ridx_v], rtok_v, idx_sem).wait()

  _pooled_side(etok_v, eemb_h, ENT_MAX_LEN, acc_v, 0, bufs, sems)
  _pooled_side(rtok_v, remb_h, REL_MAX_LEN, acc_v, DIM, bufs, sems)

  pltpu.sync_copy(acc_v, out_h.at[pl.ds(base, BW)])


@jax.jit
def kernel(subj, rel, entity_token_ids, relation_token_ids,
           entity_emb, relation_emb):
  mesh = plsc.VectorSubcoreMesh(core_axis_name="c", subcore_axis_name="s")
  run = pl.kernel(
      _body,
      out_type=jax.ShapeDtypeStruct((BATCH, 2 * DIM), jnp.float32),
      mesh=mesh,
      scratch_types=[
          pltpu.VMEM((BW,), jnp.int32),
          pltpu.VMEM((BW,), jnp.int32),
          pltpu.VMEM((BW, ENT_MAX_LEN), jnp.int32),
          pltpu.VMEM((BW, REL_MAX_LEN), jnp.int32),
          pltpu.VMEM((BW, 2 * DIM), jnp.float32),
          [pltpu.VMEM((ENT_MAX_LEN, DIM), jnp.float32) for _ in range(NBUF)],
          [pltpu.SemaphoreType.DMA for _ in range(NBUF)],
          pltpu.SemaphoreType.DMA,
      ],
  )
  return run(subj, rel, entity_token_ids, relation_token_ids,
             entity_emb, relation_emb)


# speedup vs baseline: 1.9336x; 1.9336x over previous
import jax
import jax.numpy as jnp
from jax import lax
from jax.experimental import pallas as pl
from jax.experimental.pallas import tpu as pltpu
from jax.experimental.pallas import tpu_sc as plsc

ENT_MAX_LEN = 20
REL_MAX_LEN = 20
DIM = 128
BATCH = 4096

NUM_CORES = 2
NUM_SUBCORES = 16
NW = NUM_CORES * NUM_SUBCORES
BW = BATCH // NW
NBUF = 6
L = 16


def _clamp_ids(ref_1d, n, hi):
  for c in range(n // L):
    v = ref_1d[pl.ds(c * L, L)]
    ref_1d[pl.ds(c * L, L)] = jnp.minimum(jnp.maximum(v, 0), hi)


def _clamp_tok(tok, n_tok, hi):
  for off in (0, n_tok - L):
    v = tok[pl.ds(off, L)]
    tok[pl.ds(off, L)] = jnp.minimum(jnp.maximum(v, 0), hi)


def _accum_elem(rows, n_tok, acc_v, i, col0):
  for c in range(DIM // L):
    s = rows[0, pl.ds(c * L, L)]
    for t in range(1, n_tok):
      s = s + rows[t, pl.ds(c * L, L)]
    acc_v[i, pl.ds(col0 + c * L, L)] = s


def _side(tok_h, emb_h, idx_s, n_tok, hi_tok, acc_v, col0,
          toks, rowss, tsems, rsems):
  for b in range(NBUF):
    pltpu.async_copy(tok_h.at[idx_s[b]], toks[b], tsems[b])

  n_grp = BW // NBUF

  def grp(g, _):
    i0 = g * NBUF
    for b in range(NBUF):
      i = i0 + b
      pltpu.make_async_copy(tok_h.at[idx_s[i]], toks[b], tsems[b]).wait()
      _clamp_tok(toks[b], n_tok, hi_tok)
      pltpu.async_copy(emb_h.at[toks[b]], rowss[b], rsems[b])
    for b in range(NBUF):
      i = i0 + b
      pltpu.make_async_copy(emb_h.at[toks[b]], rowss[b], rsems[b]).wait()
      _accum_elem(rowss[b], n_tok, acc_v, i, col0)
      nxt = i + NBUF

      @pl.when(nxt < BW)
      def _():
        pltpu.async_copy(tok_h.at[idx_s[nxt]], toks[b], tsems[b])
    return 0

  lax.fori_loop(0, n_grp, grp, 0)

  tail = BW % NBUF
  for b in range(tail):
    i = (BW // NBUF) * NBUF + b
    pltpu.make_async_copy(tok_h.at[idx_s[i]], toks[b], tsems[b]).wait()
    _clamp_tok(toks[b], n_tok, hi_tok)
    pltpu.async_copy(emb_h.at[toks[b]], rowss[b], rsems[b]).wait()
    _accum_elem(rowss[b], n_tok, acc_v, i, col0)


def _body(subj_h, rel_h, etok_h, rtok_h, eemb_h, remb_h, out_h,
          ids_sh, sidx_v, ridx_v, sidx_s, ridx_s, acc_v,
          toks, rowss, tsems, rsems):
  c = lax.axis_index("c")
  s = lax.axis_index("s")
  wid = s * NUM_CORES + c
  base = wid * BW

  pltpu.sync_copy(subj_h.at[pl.ds(base, BW)], sidx_v)
  pltpu.sync_copy(rel_h.at[pl.ds(base, BW)], ridx_v)
  _clamp_ids(sidx_v, BW, 100000 - 1)
  _clamp_ids(ridx_v, BW, 1000 - 1)
  pltpu.sync_copy(sidx_v, ids_sh.at[s, 0])
  pltpu.sync_copy(ridx_v, ids_sh.at[s, 1])
  pltpu.sync_copy(ids_sh.at[s, 0], sidx_s)
  pltpu.sync_copy(ids_sh.at[s, 1], ridx_s)

  _side(etok_h, eemb_h, sidx_s, ENT_MAX_LEN, 100000 - 1, acc_v, 0,
        toks, rowss, tsems, rsems)
  _side(rtok_h, remb_h, ridx_s, REL_MAX_LEN, 1000 - 1, acc_v, DIM,
        toks, rowss, tsems, rsems)

  pltpu.sync_copy(acc_v, out_h.at[pl.ds(base, BW)])


@jax.jit
def kernel(subj, rel, entity_token_ids, relation_token_ids,
           entity_emb, relation_emb):
  mesh = plsc.VectorSubcoreMesh(core_axis_name="c", subcore_axis_name="s")
  run = pl.kernel(
      _body,
      out_type=jax.ShapeDtypeStruct((BATCH, 2 * DIM), jnp.float32),
      mesh=mesh,
      scratch_types=[
          pltpu.VMEM_SHARED((NUM_SUBCORES, 2, BW), jnp.int32),
          pltpu.VMEM((BW,), jnp.int32),
          pltpu.VMEM((BW,), jnp.int32),
          pltpu.SMEM((BW,), jnp.int32),
          pltpu.SMEM((BW,), jnp.int32),
          pltpu.VMEM((BW, 2 * DIM), jnp.float32),
          [pltpu.VMEM((ENT_MAX_LEN,), jnp.int32) for _ in range(NBUF)],
          [pltpu.VMEM((ENT_MAX_LEN, DIM), jnp.float32) for _ in range(NBUF)],
          [pltpu.SemaphoreType.DMA for _ in range(NBUF)],
          [pltpu.SemaphoreType.DMA for _ in range(NBUF)],
      ],
  )
  return run(subj, rel, entity_token_ids, relation_token_ids,
             entity_emb, relation_emb)
